# baseline (speedup 1.0000x reference)
import functools

import jax
import jax.numpy as jnp
from jax.experimental import pallas as pl


def _chamfer_body(x1_ref, x2t_ref, d1_ref, d2_ref, *, num_mb):
    mb = pl.program_id(1)
    x1 = x1_ref[0]
    x2t = x2t_ref[0]
    x1sq = jnp.sum(x1 * x1, axis=1, keepdims=True)
    x2sq = jnp.sum(x2t * x2t, axis=0, keepdims=True)
    xy = jax.lax.dot_general(
        x1, x2t, (((1,), (0,)), ((), ())),
        preferred_element_type=jnp.float32,
    )
    d = (x2sq - (xy + xy)) + x1sq
    part1 = jnp.min(d, axis=1)

    @pl.when(mb == 0)
    def _():
        d1_ref[0, 0] = part1

    @pl.when(mb > 0)
    def _():
        d1_ref[0, 0] = jnp.minimum(d1_ref[0, 0], part1)

    d2_ref[0, 0] = jnp.min(d, axis=0)


def _chamfer_dists(xyz1, xyz2, *, tm=512, interpret=False):
    B, N, _ = xyz1.shape
    M = xyz2.shape[1]
    num_mb = M // tm
    x2t = jnp.transpose(xyz2, (0, 2, 1))

    d1, d2 = pl.pallas_call(
        functools.partial(_chamfer_body, num_mb=num_mb),
        grid=(B, num_mb),
        in_specs=[
            pl.BlockSpec((1, N, 3), lambda b, mb: (b, 0, 0)),
            pl.BlockSpec((1, 3, tm), lambda b, mb: (b, 0, mb)),
        ],
        out_specs=[
            pl.BlockSpec((1, 1, N), lambda b, mb: (b, 0, 0)),
            pl.BlockSpec((1, 1, tm), lambda b, mb: (b, 0, mb)),
        ],
        out_shape=[
            jax.ShapeDtypeStruct((B, 1, N), jnp.float32),
            jax.ShapeDtypeStruct((B, 1, M), jnp.float32),
        ],
        interpret=interpret,
    )(xyz1, x2t)
    return d1[:, 0, :], d2[:, 0, :]


@jax.jit
def kernel(xyz1, xyz2, weights1, weights2):
    dist1, dist2 = _chamfer_dists(xyz1, xyz2)
    dist1_avg = jnp.sum(dist1 * weights1) / jnp.sum(weights1)
    dist2_avg = jnp.sum(dist2 * weights2) / jnp.sum(weights2)
    return (dist1_avg + dist2_avg) / 2.0


# speedup vs baseline: 1.0859x; 1.0859x over previous
import functools

import jax
import jax.numpy as jnp
from jax.experimental import pallas as pl


def _chamfer_body(x1_ref, x2t_ref, d1_ref, d2_ref, *, num_mb):
    mb = pl.program_id(1)
    a = x1_ref[0]
    bt = x2t_ref[0]
    d = jax.lax.dot_general(
        a, bt, (((1,), (0,)), ((), ())),
        preferred_element_type=jnp.float32,
    )
    part1 = jnp.min(d, axis=1)

    @pl.when(mb == 0)
    def _():
        d1_ref[0, 0] = part1

    @pl.when(mb > 0)
    def _():
        d1_ref[0, 0] = jnp.minimum(d1_ref[0, 0], part1)

    d2_ref[0, 0] = jnp.min(d, axis=0)


def _chamfer_dists(xyz1, xyz2, *, tm=512, interpret=False):
    B, N, _ = xyz1.shape
    M = xyz2.shape[1]
    num_mb = M // tm
    ones_n = jnp.ones((B, N, 1), jnp.float32)
    x1sq = jnp.sum(xyz1 * xyz1, axis=2, keepdims=True)
    a = jnp.concatenate([-2.0 * xyz1, ones_n, x1sq], axis=2)
    x2t = jnp.transpose(xyz2, (0, 2, 1))
    x2sq = jnp.sum(x2t * x2t, axis=1, keepdims=True)
    ones_m = jnp.ones((B, 1, M), jnp.float32)
    bt = jnp.concatenate([x2t, x2sq, ones_m], axis=1)

    d1, d2 = pl.pallas_call(
        functools.partial(_chamfer_body, num_mb=num_mb),
        grid=(B, num_mb),
        in_specs=[
            pl.BlockSpec((1, N, 5), lambda b, mb: (b, 0, 0)),
            pl.BlockSpec((1, 5, tm), lambda b, mb: (b, 0, mb)),
        ],
        out_specs=[
            pl.BlockSpec((1, 1, N), lambda b, mb: (b, 0, 0)),
            pl.BlockSpec((1, 1, tm), lambda b, mb: (b, 0, mb)),
        ],
        out_shape=[
            jax.ShapeDtypeStruct((B, 1, N), jnp.float32),
            jax.ShapeDtypeStruct((B, 1, M), jnp.float32),
        ],
        interpret=interpret,
    )(a, bt)
    return d1[:, 0, :], d2[:, 0, :]


@jax.jit
def kernel(xyz1, xyz2, weights1, weights2):
    dist1, dist2 = _chamfer_dists(xyz1, xyz2)
    dist1_avg = jnp.sum(dist1 * weights1) / jnp.sum(weights1)
    dist2_avg = jnp.sum(dist2 * weights2) / jnp.sum(weights2)
    return (dist1_avg + dist2_avg) / 2.0
